# baseline (speedup 1.0000x reference)
import functools

import jax
import jax.numpy as jnp
import numpy as np
from jax import lax
from jax.experimental import pallas as pl
from jax.experimental.pallas import tpu as pltpu
from jax.experimental.pallas import tpu_sc as plsc

N = 10000
D = 128
E = 320000

NC = 2
NS = 16
L = 16
NW = NC * NS

NP = 10240
EC = 128
NCH = 80
NCHP = NCH + 2
E3 = NW * NCHP * EC
ROWS_T = NP // NS

BM = 256

_I0 = np.int32(0)


def _new_mesh():
    return plsc.VectorSubcoreMesh(core_axis_name="c", subcore_axis_name="s")


@functools.partial(
    pl.kernel,
    out_type=jax.ShapeDtypeStruct((NC, NP, D), jnp.float32),
    mesh=_new_mesh(),
    scratch_types=[
        pltpu.VMEM_SHARED((NP, D), jnp.float32),
        pltpu.VMEM((EC, D), jnp.float32),
        pltpu.VMEM((EC,), jnp.int32),
        pltpu.VMEM((EC,), jnp.int32),
        pltpu.SemaphoreType.DMA,
    ],
)
def _sc_degree(dst_hbm, zeros_hbm, ones_hbm, out_hbm, deg_sh, ones_v, da, db, isem):
    c = lax.axis_index("c")
    s = lax.axis_index("s")
    w = s * NC + c

    pltpu.sync_copy(zeros_hbm, deg_sh.at[pl.ds(s * ROWS_T, ROWS_T)])
    pltpu.sync_copy(ones_hbm, ones_v)
    plsc.subcore_barrier()

    base0 = w * (NCHP * EC)

    def drain_i(buf):
        pltpu.make_async_copy(dst_hbm.at[pl.ds(jnp.int32(0), EC)], buf, isem).wait()

    pltpu.sync_copy(dst_hbm.at[pl.ds(base0, EC)], da)
    pltpu.async_copy(dst_hbm.at[pl.ds(base0 + EC, EC)], db, isem)

    def pair(t, carry):
        e = 2 * t
        drain_i(db)
        pltpu.sync_copy(ones_v, deg_sh.at[da], add=True)
        pltpu.async_copy(dst_hbm.at[pl.ds(base0 + (e + 2) * EC, EC)], da, isem)
        pltpu.sync_copy(ones_v, deg_sh.at[db], add=True)
        pltpu.async_copy(dst_hbm.at[pl.ds(base0 + (e + 3) * EC, EC)], db, isem)
        drain_i(da)
        return carry

    lax.fori_loop(jnp.int32(0), jnp.int32(NCH // 2), pair, jnp.int32(0))
    drain_i(db)
    plsc.subcore_barrier()

    pltpu.sync_copy(deg_sh.at[pl.ds(s * ROWS_T, ROWS_T)],
                    out_hbm.at[c].at[pl.ds(s * ROWS_T, ROWS_T)])


@functools.partial(
    pl.kernel,
    out_type=jax.ShapeDtypeStruct((NC, NP, D), jnp.float32),
    mesh=_new_mesh(),
    scratch_types=[
        pltpu.VMEM_SHARED((NP, D), jnp.float32),
        pltpu.VMEM((EC,), jnp.int32),
        pltpu.VMEM((EC,), jnp.int32),
        pltpu.VMEM((EC,), jnp.int32),
        pltpu.VMEM((EC,), jnp.int32),
        pltpu.VMEM((EC, D), jnp.float32),
        pltpu.VMEM((EC, D), jnp.float32),
        pltpu.SemaphoreType.DMA,
        pltpu.SemaphoreType.DMA,
    ],
)
def _sc_segsum(xs_hbm, src_hbm, dst_hbm, zrows_hbm, out_hbm,
               agg_sh, sa, sb, da, db, bufa, bufb, isem, gsem):
    c = lax.axis_index("c")
    s = lax.axis_index("s")
    w = s * NC + c

    pltpu.sync_copy(zrows_hbm, agg_sh.at[pl.ds(s * ROWS_T, ROWS_T)])
    plsc.subcore_barrier()

    base0 = w * (NCHP * EC)

    def drain_i(buf):
        pltpu.make_async_copy(src_hbm.at[pl.ds(jnp.int32(0), EC)], buf, isem).wait()

    def drain_g(buf):
        pltpu.make_async_copy(xs_hbm.at[pl.ds(jnp.int32(0), EC)], buf, gsem).wait()

    def idx_fetch(g, sbuf, dbuf):
        pltpu.async_copy(src_hbm.at[pl.ds(base0 + g * EC, EC)], sbuf, isem)
        pltpu.async_copy(dst_hbm.at[pl.ds(base0 + g * EC, EC)], dbuf, isem)

    pltpu.sync_copy(src_hbm.at[pl.ds(base0, EC)], sa)
    pltpu.sync_copy(dst_hbm.at[pl.ds(base0, EC)], da)
    pltpu.async_copy(xs_hbm.at[sa], bufa, gsem)
    idx_fetch(jnp.int32(1), sb, db)

    def pair(t, carry):
        e = 2 * t
        drain_i(sb)
        drain_i(db)
        drain_g(bufa)
        pltpu.async_copy(xs_hbm.at[sb], bufb, gsem)
        pltpu.sync_copy(bufa, agg_sh.at[da], add=True)
        idx_fetch(e + 2, sa, da)
        drain_i(sa)
        drain_i(da)
        drain_g(bufb)
        pltpu.async_copy(xs_hbm.at[sa], bufa, gsem)
        pltpu.sync_copy(bufb, agg_sh.at[db], add=True)
        idx_fetch(e + 3, sb, db)
        return carry

    lax.fori_loop(jnp.int32(0), jnp.int32(NCH // 2), pair, jnp.int32(0))
    drain_i(sb)
    drain_i(db)
    drain_g(bufa)
    plsc.subcore_barrier()

    pltpu.sync_copy(agg_sh.at[pl.ds(s * ROWS_T, ROWS_T)],
                    out_hbm.at[c].at[pl.ds(s * ROWS_T, ROWS_T)])


def _tc_prep_body(d0_ref, d1_ref, x_ref, xs_ref, dv_ref):
    deg = d0_ref[...][:, 0:1] + d1_ref[...][:, 0:1]
    dv = lax.rsqrt(deg + 1.0)
    xs_ref[...] = x_ref[...] * dv
    dv_ref[...] = dv


def _tc_prep(d0, d1, x_p):
    return pl.pallas_call(
        _tc_prep_body,
        grid=(NP // BM,),
        in_specs=[
            pl.BlockSpec((BM, D), lambda i: (i, _I0)),
            pl.BlockSpec((BM, D), lambda i: (i, _I0)),
            pl.BlockSpec((BM, D), lambda i: (i, _I0)),
        ],
        out_specs=[
            pl.BlockSpec((BM, D), lambda i: (i, _I0)),
            pl.BlockSpec((BM, 1), lambda i: (i, _I0)),
        ],
        out_shape=[
            jax.ShapeDtypeStruct((NP, D), jnp.float32),
            jax.ShapeDtypeStruct((NP, 1), jnp.float32),
        ],
    )(d0, d1, x_p)


def _tc_finish_body(x_ref, xs_ref, a0_ref, a1_ref, dv_ref, w_ref,
                    b_ref, g_ref, bt_ref, o_ref):
    sagg = a0_ref[...] + a1_ref[...] + xs_ref[...]
    h = jnp.dot(sagg, w_ref[...], preferred_element_type=jnp.float32)
    y = x_ref[...] + dv_ref[...] * h + b_ref[...]
    mu = jnp.mean(y, axis=-1, keepdims=True)
    yc = y - mu
    var = jnp.mean(yc * yc, axis=-1, keepdims=True)
    o_ref[...] = yc * lax.rsqrt(var + 1e-3) * g_ref[...] + bt_ref[...]


def _tc_finish(x_p, xs, a0, a1, dv, W, b, gamma, beta):
    return pl.pallas_call(
        _tc_finish_body,
        grid=(NP // BM,),
        in_specs=[
            pl.BlockSpec((BM, D), lambda i: (i, _I0)),
            pl.BlockSpec((BM, D), lambda i: (i, _I0)),
            pl.BlockSpec((BM, D), lambda i: (i, _I0)),
            pl.BlockSpec((BM, D), lambda i: (i, _I0)),
            pl.BlockSpec((BM, 1), lambda i: (i, _I0)),
            pl.BlockSpec((D, D), lambda i: (_I0, _I0)),
            pl.BlockSpec((1, D), lambda i: (_I0, _I0)),
            pl.BlockSpec((1, D), lambda i: (_I0, _I0)),
            pl.BlockSpec((1, D), lambda i: (_I0, _I0)),
        ],
        out_specs=pl.BlockSpec((BM, D), lambda i: (i, _I0)),
        out_shape=jax.ShapeDtypeStruct((NP, D), jnp.float32),
    )(x_p, xs, a0, a1, dv, W, b, gamma, beta)


def kernel(x, edge_index, W, b, gamma, beta):
    src = edge_index[0].astype(jnp.int32)
    dst = edge_index[1].astype(jnp.int32)
    x_p = jnp.pad(x.astype(jnp.float32), ((0, NP - N), (0, 0)))

    def _blocks(a):
        t = jnp.full((NW * NCH * EC - E,), NP - 1, jnp.int32)
        real = jnp.concatenate([a, t]).reshape(NW, NCH, EC)
        pad = jnp.full((NW, NCHP - NCH, EC), NP - 1, jnp.int32)
        return jnp.concatenate([real, pad], axis=1).reshape(-1)

    src_p = _blocks(src)
    dst_p = _blocks(dst)

    zeros_d = jnp.zeros((ROWS_T, D), jnp.float32)
    ones_d = jnp.ones((EC, D), jnp.float32)

    deg = _sc_degree(dst_p, zeros_d, ones_d)
    xs, dv = _tc_prep(deg[0], deg[1], x_p)
    agg = _sc_segsum(xs, src_p, dst_p, zeros_d)
    out = _tc_finish(x_p, xs, agg[0], agg[1], dv,
                     W.astype(jnp.float32),
                     b.astype(jnp.float32).reshape(1, D),
                     gamma.astype(jnp.float32).reshape(1, D),
                     beta.astype(jnp.float32).reshape(1, D))
    return out[:N]


# speedup vs baseline: 1.1411x; 1.1411x over previous
import functools

import jax
import jax.numpy as jnp
import numpy as np
from jax import lax
from jax.experimental import pallas as pl
from jax.experimental.pallas import tpu as pltpu
from jax.experimental.pallas import tpu_sc as plsc

N = 10000
D = 128
E = 320000

NC = 2
NS = 16
L = 16
NW = NC * NS

NP = 10240
EC = 128
NCH = 80
NCHP = NCH + 2
E3 = NW * NCHP * EC
ROWS_T = NP // NS

BM = 256

_I0 = np.int32(0)


def _new_mesh():
    return plsc.VectorSubcoreMesh(core_axis_name="c", subcore_axis_name="s")


@functools.partial(
    pl.kernel,
    out_type=jax.ShapeDtypeStruct((NC, NP, D), jnp.float32),
    mesh=_new_mesh(),
    scratch_types=[
        pltpu.VMEM_SHARED((NP, D), jnp.float32),
        pltpu.VMEM((EC, D), jnp.float32),
        pltpu.VMEM((EC,), jnp.int32),
        pltpu.VMEM((EC,), jnp.int32),
        pltpu.SemaphoreType.DMA,
    ],
)
def _sc_degree(dst_hbm, zeros_hbm, ones_hbm, out_hbm, deg_sh, ones_v, da, db, isem):
    c = lax.axis_index("c")
    s = lax.axis_index("s")
    w = s * NC + c

    pltpu.sync_copy(zeros_hbm, deg_sh.at[pl.ds(s * ROWS_T, ROWS_T)])
    pltpu.sync_copy(ones_hbm, ones_v)
    plsc.subcore_barrier()

    base0 = w * (NCHP * EC)

    def drain_i(buf):
        pltpu.make_async_copy(dst_hbm.at[pl.ds(jnp.int32(0), EC)], buf, isem).wait()

    pltpu.sync_copy(dst_hbm.at[pl.ds(base0, EC)], da)
    pltpu.async_copy(dst_hbm.at[pl.ds(base0 + EC, EC)], db, isem)

    def pair(t, carry):
        e = 2 * t
        drain_i(db)
        pltpu.sync_copy(ones_v, deg_sh.at[da], add=True)
        pltpu.async_copy(dst_hbm.at[pl.ds(base0 + (e + 2) * EC, EC)], da, isem)
        pltpu.sync_copy(ones_v, deg_sh.at[db], add=True)
        pltpu.async_copy(dst_hbm.at[pl.ds(base0 + (e + 3) * EC, EC)], db, isem)
        drain_i(da)
        return carry

    lax.fori_loop(jnp.int32(0), jnp.int32(NCH // 2), pair, jnp.int32(0))
    drain_i(db)
    plsc.subcore_barrier()

    pltpu.sync_copy(deg_sh.at[pl.ds(s * ROWS_T, ROWS_T)],
                    out_hbm.at[c].at[pl.ds(s * ROWS_T, ROWS_T)])


@functools.partial(
    pl.kernel,
    out_type=jax.ShapeDtypeStruct((NC, NP, D), jnp.float32),
    mesh=_new_mesh(),
    scratch_types=[
        pltpu.VMEM_SHARED((NP, D), jnp.float32),
        pltpu.VMEM((EC,), jnp.int32),
        pltpu.VMEM((EC,), jnp.int32),
        pltpu.VMEM((EC,), jnp.int32),
        pltpu.VMEM((EC,), jnp.int32),
        pltpu.VMEM((EC, D), jnp.float32),
        pltpu.VMEM((EC, D), jnp.float32),
        pltpu.SemaphoreType.DMA,
        pltpu.SemaphoreType.DMA,
    ],
)
def _sc_segsum(xs_hbm, src_hbm, dst_hbm, zrows_hbm, out_hbm,
               agg_sh, sa, sb, da, db, bufa, bufb, gsem, ssem):
    c = lax.axis_index("c")
    s = lax.axis_index("s")
    w = s * NC + c

    pltpu.sync_copy(zrows_hbm, agg_sh.at[pl.ds(s * ROWS_T, ROWS_T)])
    plsc.subcore_barrier()

    base0 = w * (NCHP * EC)

    def drain_s(buf):
        pltpu.make_async_copy(xs_hbm.at[pl.ds(jnp.int32(0), EC)], buf, ssem).wait()

    def idx_load(g, sbuf, dbuf):
        pltpu.sync_copy(src_hbm.at[pl.ds(base0 + g * EC, EC)], sbuf)
        pltpu.sync_copy(dst_hbm.at[pl.ds(base0 + g * EC, EC)], dbuf)

    idx_load(jnp.int32(0), sa, da)

    def pair(t, carry):
        e = 2 * t
        pltpu.async_copy(xs_hbm.at[sa], bufa, gsem).wait()

        @pl.when(t > 0)
        def _():
            drain_s(bufb)

        pltpu.async_copy(bufa, agg_sh.at[da], ssem, add=True)
        idx_load(e + 1, sb, db)
        pltpu.async_copy(xs_hbm.at[sb], bufb, gsem).wait()
        drain_s(bufa)
        pltpu.async_copy(bufb, agg_sh.at[db], ssem, add=True)
        idx_load(e + 2, sa, da)
        return carry

    lax.fori_loop(jnp.int32(0), jnp.int32(NCH // 2), pair, jnp.int32(0))
    drain_s(bufb)
    plsc.subcore_barrier()

    pltpu.sync_copy(agg_sh.at[pl.ds(s * ROWS_T, ROWS_T)],
                    out_hbm.at[c].at[pl.ds(s * ROWS_T, ROWS_T)])


def _tc_prep_body(d0_ref, d1_ref, x_ref, xs_ref, dv_ref):
    deg = d0_ref[...][:, 0:1] + d1_ref[...][:, 0:1]
    dv = lax.rsqrt(deg + 1.0)
    xs_ref[...] = x_ref[...] * dv
    dv_ref[...] = dv


def _tc_prep(d0, d1, x_p):
    return pl.pallas_call(
        _tc_prep_body,
        grid=(NP // BM,),
        in_specs=[
            pl.BlockSpec((BM, D), lambda i: (i, _I0)),
            pl.BlockSpec((BM, D), lambda i: (i, _I0)),
            pl.BlockSpec((BM, D), lambda i: (i, _I0)),
        ],
        out_specs=[
            pl.BlockSpec((BM, D), lambda i: (i, _I0)),
            pl.BlockSpec((BM, 1), lambda i: (i, _I0)),
        ],
        out_shape=[
            jax.ShapeDtypeStruct((NP, D), jnp.float32),
            jax.ShapeDtypeStruct((NP, 1), jnp.float32),
        ],
    )(d0, d1, x_p)


def _tc_finish_body(x_ref, xs_ref, a0_ref, a1_ref, dv_ref, w_ref,
                    b_ref, g_ref, bt_ref, o_ref):
    sagg = a0_ref[...] + a1_ref[...] + xs_ref[...]
    h = jnp.dot(sagg, w_ref[...], preferred_element_type=jnp.float32)
    y = x_ref[...] + dv_ref[...] * h + b_ref[...]
    mu = jnp.mean(y, axis=-1, keepdims=True)
    yc = y - mu
    var = jnp.mean(yc * yc, axis=-1, keepdims=True)
    o_ref[...] = yc * lax.rsqrt(var + 1e-3) * g_ref[...] + bt_ref[...]


def _tc_finish(x_p, xs, a0, a1, dv, W, b, gamma, beta):
    return pl.pallas_call(
        _tc_finish_body,
        grid=(NP // BM,),
        in_specs=[
            pl.BlockSpec((BM, D), lambda i: (i, _I0)),
            pl.BlockSpec((BM, D), lambda i: (i, _I0)),
            pl.BlockSpec((BM, D), lambda i: (i, _I0)),
            pl.BlockSpec((BM, D), lambda i: (i, _I0)),
            pl.BlockSpec((BM, 1), lambda i: (i, _I0)),
            pl.BlockSpec((D, D), lambda i: (_I0, _I0)),
            pl.BlockSpec((1, D), lambda i: (_I0, _I0)),
            pl.BlockSpec((1, D), lambda i: (_I0, _I0)),
            pl.BlockSpec((1, D), lambda i: (_I0, _I0)),
        ],
        out_specs=pl.BlockSpec((BM, D), lambda i: (i, _I0)),
        out_shape=jax.ShapeDtypeStruct((NP, D), jnp.float32),
    )(x_p, xs, a0, a1, dv, W, b, gamma, beta)


def kernel(x, edge_index, W, b, gamma, beta):
    src = edge_index[0].astype(jnp.int32)
    dst = edge_index[1].astype(jnp.int32)
    x_p = jnp.pad(x.astype(jnp.float32), ((0, NP - N), (0, 0)))

    def _blocks(a):
        t = jnp.full((NW * NCH * EC - E,), NP - 1, jnp.int32)
        real = jnp.concatenate([a, t]).reshape(NW, NCH, EC)
        pad = jnp.full((NW, NCHP - NCH, EC), NP - 1, jnp.int32)
        return jnp.concatenate([real, pad], axis=1).reshape(-1)

    src_p = _blocks(src)
    dst_p = _blocks(dst)

    zeros_d = jnp.zeros((ROWS_T, D), jnp.float32)
    ones_d = jnp.ones((EC, D), jnp.float32)

    deg = _sc_degree(dst_p, zeros_d, ones_d)
    xs, dv = _tc_prep(deg[0], deg[1], x_p)
    agg = _sc_segsum(xs, src_p, dst_p, zeros_d)
    out = _tc_finish(x_p, xs, agg[0], agg[1], dv,
                     W.astype(jnp.float32),
                     b.astype(jnp.float32).reshape(1, D),
                     gamma.astype(jnp.float32).reshape(1, D),
                     beta.astype(jnp.float32).reshape(1, D))
    return out[:N]


# speedup vs baseline: 1.1609x; 1.0174x over previous
import functools

import jax
import jax.numpy as jnp
import numpy as np
from jax import lax
from jax.experimental import pallas as pl
from jax.experimental.pallas import tpu as pltpu
from jax.experimental.pallas import tpu_sc as plsc

N = 10000
D = 128
E = 320000

NC = 2
NS = 16
L = 16
NW = NC * NS

NP = 10240
EC = 128
NCH = 80
NCHP = NCH + 2
E3 = NW * NCHP * EC
ROWS_T = NP // NS

BM = 256

_I0 = np.int32(0)


def _new_mesh():
    return plsc.VectorSubcoreMesh(core_axis_name="c", subcore_axis_name="s")


@functools.partial(
    pl.kernel,
    out_type=jax.ShapeDtypeStruct((NC, NP, D), jnp.float32),
    mesh=_new_mesh(),
    scratch_types=[
        pltpu.VMEM_SHARED((NP, D), jnp.float32),
        pltpu.VMEM((EC, D), jnp.float32),
        pltpu.VMEM((EC,), jnp.int32),
        pltpu.VMEM((EC,), jnp.int32),
        pltpu.SemaphoreType.DMA,
    ],
)
def _sc_degree(dst_hbm, zeros_hbm, ones_hbm, out_hbm, deg_sh, ones_v, da, db, isem):
    c = lax.axis_index("c")
    s = lax.axis_index("s")
    w = s * NC + c

    pltpu.sync_copy(zeros_hbm, deg_sh.at[pl.ds(s * ROWS_T, ROWS_T)])
    pltpu.sync_copy(ones_hbm, ones_v)
    plsc.subcore_barrier()

    base0 = w * (NCHP * EC)

    def drain_i(buf):
        pltpu.make_async_copy(dst_hbm.at[pl.ds(jnp.int32(0), EC)], buf, isem).wait()

    pltpu.sync_copy(dst_hbm.at[pl.ds(base0, EC)], da)
    pltpu.async_copy(dst_hbm.at[pl.ds(base0 + EC, EC)], db, isem)

    def pair(t, carry):
        e = 2 * t
        drain_i(db)
        pltpu.sync_copy(ones_v, deg_sh.at[da], add=True)
        pltpu.async_copy(dst_hbm.at[pl.ds(base0 + (e + 2) * EC, EC)], da, isem)
        pltpu.sync_copy(ones_v, deg_sh.at[db], add=True)
        pltpu.async_copy(dst_hbm.at[pl.ds(base0 + (e + 3) * EC, EC)], db, isem)
        drain_i(da)
        return carry

    lax.fori_loop(jnp.int32(0), jnp.int32(NCH // 2), pair, jnp.int32(0))
    drain_i(db)
    plsc.subcore_barrier()

    pltpu.sync_copy(deg_sh.at[pl.ds(s * ROWS_T, ROWS_T)],
                    out_hbm.at[c].at[pl.ds(s * ROWS_T, ROWS_T)])


@functools.partial(
    pl.kernel,
    out_type=jax.ShapeDtypeStruct((NC, NP, D), jnp.float32),
    mesh=_new_mesh(),
    scratch_types=[
        pltpu.VMEM_SHARED((NP, D), jnp.float32),
        pltpu.VMEM((EC,), jnp.int32),
        pltpu.VMEM((EC,), jnp.int32),
        pltpu.VMEM((EC,), jnp.int32),
        pltpu.VMEM((EC,), jnp.int32),
        pltpu.VMEM((EC, D), jnp.float32),
        pltpu.VMEM((EC, D), jnp.float32),
        pltpu.SemaphoreType.DMA,
        pltpu.SemaphoreType.DMA,
    ],
)
def _sc_segsum(xs_hbm, src_hbm, dst_hbm, zrows_hbm, out_hbm,
               agg_sh, sa, sb, da, db, bufa, bufb, gsem, isem):
    c = lax.axis_index("c")
    s = lax.axis_index("s")
    w = s * NC + c

    pltpu.sync_copy(zrows_hbm, agg_sh.at[pl.ds(s * ROWS_T, ROWS_T)])
    plsc.subcore_barrier()

    base0 = w * (NCHP * EC)

    idx_load_a = lambda g: (
        pltpu.async_copy(src_hbm.at[pl.ds(base0 + g * EC, EC)], sa, isem),
        pltpu.async_copy(dst_hbm.at[pl.ds(base0 + g * EC, EC)], da, isem))
    idx_load_b = lambda g: (
        pltpu.async_copy(src_hbm.at[pl.ds(base0 + g * EC, EC)], sb, isem),
        pltpu.async_copy(dst_hbm.at[pl.ds(base0 + g * EC, EC)], db, isem))

    def drain_i(buf):
        pltpu.make_async_copy(src_hbm.at[pl.ds(jnp.int32(0), EC)], buf, isem).wait()

    pltpu.sync_copy(src_hbm.at[pl.ds(base0, EC)], sa)
    pltpu.sync_copy(dst_hbm.at[pl.ds(base0, EC)], da)
    idx_load_b(jnp.int32(1))

    def pair(t, carry):
        e = 2 * t
        pltpu.async_copy(xs_hbm.at[sa], bufa, gsem).wait()
        pltpu.sync_copy(bufa, agg_sh.at[da], add=True)
        idx_load_a(e + 2)
        drain_i(sb)
        drain_i(db)
        pltpu.async_copy(xs_hbm.at[sb], bufb, gsem).wait()
        pltpu.sync_copy(bufb, agg_sh.at[db], add=True)
        idx_load_b(e + 3)
        drain_i(sa)
        drain_i(da)
        return carry

    lax.fori_loop(jnp.int32(0), jnp.int32(NCH // 2), pair, jnp.int32(0))
    drain_i(sb)
    drain_i(db)
    plsc.subcore_barrier()

    pltpu.sync_copy(agg_sh.at[pl.ds(s * ROWS_T, ROWS_T)],
                    out_hbm.at[c].at[pl.ds(s * ROWS_T, ROWS_T)])


def _tc_prep_body(d0_ref, d1_ref, x_ref, xs_ref, dv_ref):
    deg = d0_ref[...][:, 0:1] + d1_ref[...][:, 0:1]
    dv = lax.rsqrt(deg + 1.0)
    xs_ref[...] = x_ref[...] * dv
    dv_ref[...] = dv


def _tc_prep(d0, d1, x_p):
    return pl.pallas_call(
        _tc_prep_body,
        grid=(NP // BM,),
        in_specs=[
            pl.BlockSpec((BM, D), lambda i: (i, _I0)),
            pl.BlockSpec((BM, D), lambda i: (i, _I0)),
            pl.BlockSpec((BM, D), lambda i: (i, _I0)),
        ],
        out_specs=[
            pl.BlockSpec((BM, D), lambda i: (i, _I0)),
            pl.BlockSpec((BM, 1), lambda i: (i, _I0)),
        ],
        out_shape=[
            jax.ShapeDtypeStruct((NP, D), jnp.float32),
            jax.ShapeDtypeStruct((NP, 1), jnp.float32),
        ],
    )(d0, d1, x_p)


def _tc_finish_body(x_ref, xs_ref, a0_ref, a1_ref, dv_ref, w_ref,
                    b_ref, g_ref, bt_ref, o_ref):
    sagg = a0_ref[...] + a1_ref[...] + xs_ref[...]
    h = jnp.dot(sagg, w_ref[...], preferred_element_type=jnp.float32)
    y = x_ref[...] + dv_ref[...] * h + b_ref[...]
    mu = jnp.mean(y, axis=-1, keepdims=True)
    yc = y - mu
    var = jnp.mean(yc * yc, axis=-1, keepdims=True)
    o_ref[...] = yc * lax.rsqrt(var + 1e-3) * g_ref[...] + bt_ref[...]


def _tc_finish(x_p, xs, a0, a1, dv, W, b, gamma, beta):
    return pl.pallas_call(
        _tc_finish_body,
        grid=(NP // BM,),
        in_specs=[
            pl.BlockSpec((BM, D), lambda i: (i, _I0)),
            pl.BlockSpec((BM, D), lambda i: (i, _I0)),
            pl.BlockSpec((BM, D), lambda i: (i, _I0)),
            pl.BlockSpec((BM, D), lambda i: (i, _I0)),
            pl.BlockSpec((BM, 1), lambda i: (i, _I0)),
            pl.BlockSpec((D, D), lambda i: (_I0, _I0)),
            pl.BlockSpec((1, D), lambda i: (_I0, _I0)),
            pl.BlockSpec((1, D), lambda i: (_I0, _I0)),
            pl.BlockSpec((1, D), lambda i: (_I0, _I0)),
        ],
        out_specs=pl.BlockSpec((BM, D), lambda i: (i, _I0)),
        out_shape=jax.ShapeDtypeStruct((NP, D), jnp.float32),
    )(x_p, xs, a0, a1, dv, W, b, gamma, beta)


def kernel(x, edge_index, W, b, gamma, beta):
    src = edge_index[0].astype(jnp.int32)
    dst = edge_index[1].astype(jnp.int32)
    x_p = jnp.pad(x.astype(jnp.float32), ((0, NP - N), (0, 0)))

    def _blocks(a):
        t = jnp.full((NW * NCH * EC - E,), NP - 1, jnp.int32)
        real = jnp.concatenate([a, t]).reshape(NW, NCH, EC)
        pad = jnp.full((NW, NCHP - NCH, EC), NP - 1, jnp.int32)
        return jnp.concatenate([real, pad], axis=1).reshape(-1)

    src_p = _blocks(src)
    dst_p = _blocks(dst)

    zeros_d = jnp.zeros((ROWS_T, D), jnp.float32)
    ones_d = jnp.ones((EC, D), jnp.float32)

    deg = _sc_degree(dst_p, zeros_d, ones_d)
    xs, dv = _tc_prep(deg[0], deg[1], x_p)
    agg = _sc_segsum(xs, src_p, dst_p, zeros_d)
    out = _tc_finish(x_p, xs, agg[0], agg[1], dv,
                     W.astype(jnp.float32),
                     b.astype(jnp.float32).reshape(1, D),
                     gamma.astype(jnp.float32).reshape(1, D),
                     beta.astype(jnp.float32).reshape(1, D))
    return out[:N]


# speedup vs baseline: 2.1299x; 1.8346x over previous
import functools

import jax
import jax.numpy as jnp
import numpy as np
from jax import lax
from jax.experimental import pallas as pl
from jax.experimental.pallas import tpu as pltpu
from jax.experimental.pallas import tpu_sc as plsc

N = 10000
D = 128
E = 320000

NC = 2
NS = 16
L = 16
NW = NC * NS

NP = 10240
EC = 128
NCH = 80
NCHP = NCH + 2
E3 = NW * NCHP * EC
ROWS_T = NP // NS

BM = 256

_I0 = np.int32(0)


def _new_mesh():
    return plsc.VectorSubcoreMesh(core_axis_name="c", subcore_axis_name="s")


@functools.partial(
    pl.kernel,
    out_type=jax.ShapeDtypeStruct((NC, NP, D), jnp.float32),
    mesh=_new_mesh(),
    scratch_types=[
        pltpu.VMEM_SHARED((NP, D), jnp.float32),
        pltpu.VMEM((EC, D), jnp.float32),
        pltpu.VMEM((EC,), jnp.int32),
        pltpu.VMEM((EC,), jnp.int32),
        pltpu.SemaphoreType.DMA,
    ],
)
def _sc_degree(dst_hbm, zeros_hbm, ones_hbm, out_hbm, deg_sh, ones_v, da, db, isem):
    c = lax.axis_index("c")
    s = lax.axis_index("s")
    w = s * NC + c

    pltpu.sync_copy(zeros_hbm, deg_sh.at[pl.ds(s * ROWS_T, ROWS_T)])
    pltpu.sync_copy(ones_hbm, ones_v)
    plsc.subcore_barrier()

    base0 = w * (NCHP * EC)

    def drain_i(buf):
        pltpu.make_async_copy(dst_hbm.at[pl.ds(jnp.int32(0), EC)], buf, isem).wait()

    pltpu.sync_copy(dst_hbm.at[pl.ds(base0, EC)], da)
    pltpu.async_copy(dst_hbm.at[pl.ds(base0 + EC, EC)], db, isem)

    def pair(t, carry):
        e = 2 * t
        drain_i(db)
        pltpu.sync_copy(ones_v, deg_sh.at[da], add=True)
        pltpu.async_copy(dst_hbm.at[pl.ds(base0 + (e + 2) * EC, EC)], da, isem)
        pltpu.sync_copy(ones_v, deg_sh.at[db], add=True)
        pltpu.async_copy(dst_hbm.at[pl.ds(base0 + (e + 3) * EC, EC)], db, isem)
        drain_i(da)
        return carry

    lax.fori_loop(jnp.int32(0), jnp.int32(NCH // 2), pair, jnp.int32(0))
    drain_i(db)
    plsc.subcore_barrier()

    pltpu.sync_copy(deg_sh.at[pl.ds(s * ROWS_T, ROWS_T)],
                    out_hbm.at[c].at[pl.ds(s * ROWS_T, ROWS_T)])


@functools.partial(
    pl.kernel,
    out_type=jax.ShapeDtypeStruct((NC, NP, D), jnp.float32),
    mesh=_new_mesh(),
    scratch_types=[
        pltpu.VMEM_SHARED((NP, D), jnp.float32),
        pltpu.VMEM((EC,), jnp.int32),
        pltpu.VMEM((EC,), jnp.int32),
        pltpu.VMEM((EC,), jnp.int32),
        pltpu.VMEM((EC,), jnp.int32),
        pltpu.VMEM((EC, D), jnp.float32),
        pltpu.VMEM((EC, D), jnp.float32),
        pltpu.SemaphoreType.DMA,
        pltpu.SemaphoreType.DMA,
    ],
)
def _sc_segsum(xs_hbm, src_hbm, dst_hbm, zrows_hbm, out_hbm,
               agg_sh, sa, sb, da, db, bufa, bufb, gsem, isem):
    c = lax.axis_index("c")
    s = lax.axis_index("s")
    w = s * NC + c

    pltpu.sync_copy(zrows_hbm, agg_sh.at[pl.ds(s * ROWS_T, ROWS_T)])
    plsc.subcore_barrier()

    base0 = w * (NCHP * EC)

    idx_load_a = lambda g: (
        pltpu.async_copy(src_hbm.at[pl.ds(base0 + g * EC, EC)], sa, isem),
        pltpu.async_copy(dst_hbm.at[pl.ds(base0 + g * EC, EC)], da, isem))
    idx_load_b = lambda g: (
        pltpu.async_copy(src_hbm.at[pl.ds(base0 + g * EC, EC)], sb, isem),
        pltpu.async_copy(dst_hbm.at[pl.ds(base0 + g * EC, EC)], db, isem))

    def drain_i(buf):
        pltpu.make_async_copy(src_hbm.at[pl.ds(jnp.int32(0), EC)], buf, isem).wait()

    pltpu.sync_copy(src_hbm.at[pl.ds(base0, EC)], sa)
    pltpu.sync_copy(dst_hbm.at[pl.ds(base0, EC)], da)
    idx_load_b(jnp.int32(1))

    def pair(t, carry):
        e = 2 * t
        pltpu.async_copy(xs_hbm.at[sa], bufa, gsem).wait()
        pltpu.sync_copy(bufa, agg_sh.at[da], add=True)
        idx_load_a(e + 2)
        drain_i(sb)
        drain_i(db)
        pltpu.async_copy(xs_hbm.at[sb], bufb, gsem).wait()
        pltpu.sync_copy(bufb, agg_sh.at[db], add=True)
        idx_load_b(e + 3)
        drain_i(sa)
        drain_i(da)
        return carry

    lax.fori_loop(jnp.int32(0), jnp.int32(NCH // 2), pair, jnp.int32(0))
    drain_i(sb)
    drain_i(db)
    plsc.subcore_barrier()

    pltpu.sync_copy(agg_sh.at[pl.ds(s * ROWS_T, ROWS_T)],
                    out_hbm.at[c].at[pl.ds(s * ROWS_T, ROWS_T)])


def _tc_prep_body(d0_ref, d1_ref, x_ref, xs_ref, dv_ref):
    deg = d0_ref[...][:, 0:1] + d1_ref[...][:, 0:1]
    dv = lax.rsqrt(deg + 1.0)
    xs_ref[...] = x_ref[...] * dv
    dv_ref[...] = dv


def _tc_prep(d0, d1, x_p):
    return pl.pallas_call(
        _tc_prep_body,
        grid=(NP // BM,),
        in_specs=[
            pl.BlockSpec((BM, D), lambda i: (i, _I0)),
            pl.BlockSpec((BM, D), lambda i: (i, _I0)),
            pl.BlockSpec((BM, D), lambda i: (i, _I0)),
        ],
        out_specs=[
            pl.BlockSpec((BM, D), lambda i: (i, _I0)),
            pl.BlockSpec((BM, 1), lambda i: (i, _I0)),
        ],
        out_shape=[
            jax.ShapeDtypeStruct((NP, D), jnp.float32),
            jax.ShapeDtypeStruct((NP, 1), jnp.float32),
        ],
    )(d0, d1, x_p)


def _tc_finish_body(x_ref, xs_ref, a0_ref, a1_ref, dv_ref, w_ref,
                    b_ref, g_ref, bt_ref, o_ref):
    sagg = a0_ref[...] + a1_ref[...] + xs_ref[...]
    h = jnp.dot(sagg, w_ref[...], preferred_element_type=jnp.float32)
    y = x_ref[...] + dv_ref[...] * h + b_ref[...]
    mu = jnp.mean(y, axis=-1, keepdims=True)
    yc = y - mu
    var = jnp.mean(yc * yc, axis=-1, keepdims=True)
    o_ref[...] = yc * lax.rsqrt(var + 1e-3) * g_ref[...] + bt_ref[...]


def _tc_finish(x_p, xs, a0, a1, dv, W, b, gamma, beta):
    return pl.pallas_call(
        _tc_finish_body,
        grid=(NP // BM,),
        in_specs=[
            pl.BlockSpec((BM, D), lambda i: (i, _I0)),
            pl.BlockSpec((BM, D), lambda i: (i, _I0)),
            pl.BlockSpec((BM, D), lambda i: (i, _I0)),
            pl.BlockSpec((BM, D), lambda i: (i, _I0)),
            pl.BlockSpec((BM, 1), lambda i: (i, _I0)),
            pl.BlockSpec((D, D), lambda i: (_I0, _I0)),
            pl.BlockSpec((1, D), lambda i: (_I0, _I0)),
            pl.BlockSpec((1, D), lambda i: (_I0, _I0)),
            pl.BlockSpec((1, D), lambda i: (_I0, _I0)),
        ],
        out_specs=pl.BlockSpec((BM, D), lambda i: (i, _I0)),
        out_shape=jax.ShapeDtypeStruct((NP, D), jnp.float32),
    )(x_p, xs, a0, a1, dv, W, b, gamma, beta)


def kernel(x, edge_index, W, b, gamma, beta):
    src = edge_index[0].astype(jnp.int32)
    dst = edge_index[1].astype(jnp.int32)
    x_p = jnp.pad(x.astype(jnp.float32), ((0, NP - N), (0, 0)))

    epw = E // NW
    padw = NCHP * EC - epw
    padv = (N + (jnp.arange(NW * padw, dtype=jnp.int32) % (NP - N))
            ).reshape(NW, padw)

    def _blocks(a):
        return jnp.concatenate([a.reshape(NW, epw), padv], axis=1).reshape(-1)

    src_p = _blocks(src)
    dst_p = _blocks(dst)

    zeros_d = jnp.zeros((ROWS_T, D), jnp.float32)
    ones_d = jnp.ones((EC, D), jnp.float32)

    deg = _sc_degree(dst_p, zeros_d, ones_d)
    xs, dv = _tc_prep(deg[0], deg[1], x_p)
    agg = _sc_segsum(xs, src_p, dst_p, zeros_d)
    out = _tc_finish(x_p, xs, agg[0], agg[1], dv,
                     W.astype(jnp.float32),
                     b.astype(jnp.float32).reshape(1, D),
                     gamma.astype(jnp.float32).reshape(1, D),
                     beta.astype(jnp.float32).reshape(1, D))
    return out[:N]
